# baseline (speedup 1.0000x reference)
import functools

import jax
import jax.numpy as jnp
from jax import lax
from jax.experimental import pallas as pl
from jax.experimental.pallas import tpu as pltpu
from jax.experimental.pallas import tpu_sc as plsc

N = 10000
E = 320000
G = 128
D = 128
N_LAYERS = 3

NW = 32
CHUNK = 128
CH_PER_TILE = 80
E_PAD = NW * CH_PER_TILE * CHUNK
N_ACC = N + 16
ROUNDS = CH_PER_TILE

_R = 1000
_GRID = N // _R

def _sc_body(cur, packed3, out, acc, idx_v, src_a, dst_a, src_b, dst_b,
             buf_a, buf_b, sem_a, sem_b, sem_sa, sem_sb):
    cid = lax.axis_index("c")
    sid = lax.axis_index("s")
    wid = sid * 2 + cid
    rows_per_tile = 624
    tail_base = 16 * rows_per_tile
    base = sid * rows_per_tile

    pltpu.sync_copy(cur.at[pl.ds(base, rows_per_tile)],
                    acc.at[pl.ds(base, rows_per_tile)])

    @pl.when(sid == 15)
    def _():
        pltpu.sync_copy(cur.at[pl.ds(tail_base, N - tail_base)],
                        acc.at[pl.ds(tail_base, N - tail_base)])
    pltpu.sync_copy(packed3.at[wid], idx_v)
    plsc.subcore_barrier()

    def unpack(r, src_buf, dst_buf):
        for i in range(CHUNK // 16):
            v = idx_v[r, pl.ds(i * 16, 16)]
            src_buf[pl.ds(i * 16, 16)] = v & 0x3FFF
            dst_buf[pl.ds(i * 16, 16)] = lax.shift_right_logical(v, 14)

    def fire_gather(buf, sem, src_buf):
        pltpu.async_copy(cur.at[src_buf], buf, sem)

    def fire_scatter(buf, sem, dst_buf):
        pltpu.async_copy(buf, acc.at[dst_buf], sem, add=True)

    def drain(buf, sem):
        pltpu.make_async_copy(cur.at[pl.ds(0, CHUNK)], buf, sem).wait()

    unpack(0, src_a, dst_a)
    fire_gather(buf_a, sem_a, src_a)
    unpack(1, src_b, dst_b)
    fire_gather(buf_b, sem_b, src_b)

    def body(o, carry):
        r = o * 2
        drain(buf_a, sem_a)
        fire_scatter(buf_a, sem_sa, dst_a)
        drain(buf_b, sem_b)
        fire_scatter(buf_b, sem_sb, dst_b)
        drain(buf_a, sem_sa)

        @pl.when(o < ROUNDS // 2 - 1)
        def _():
            unpack(r + 2, src_a, dst_a)
            fire_gather(buf_a, sem_a, src_a)

        drain(buf_b, sem_sb)

        @pl.when(o < ROUNDS // 2 - 1)
        def _():
            unpack(r + 3, src_b, dst_b)
            fire_gather(buf_b, sem_b, src_b)
        return carry

    lax.fori_loop(0, ROUNDS // 2, body, 0)
    plsc.subcore_barrier()

    pltpu.sync_copy(acc.at[pl.ds(base, rows_per_tile)],
                    out.at[cid, pl.ds(base, rows_per_tile)])

    @pl.when(sid == 15)
    def _():
        pltpu.sync_copy(acc.at[pl.ds(tail_base, N - tail_base)],
                        out.at[cid, pl.ds(tail_base, N - tail_base)])


@functools.cache
def _sc_edge_agg_build():
    mesh = plsc.VectorSubcoreMesh(core_axis_name="c", subcore_axis_name="s")
    return pl.kernel(
        _sc_body,
        out_type=jax.ShapeDtypeStruct((2, N, D), jnp.float32),
        mesh=mesh,
        scratch_types=[
            pltpu.VMEM_SHARED((N_ACC, D), jnp.float32),
            pltpu.VMEM((CH_PER_TILE, CHUNK), jnp.int32),
            pltpu.VMEM((CHUNK,), jnp.int32),
            pltpu.VMEM((CHUNK,), jnp.int32),
            pltpu.VMEM((CHUNK,), jnp.int32),
            pltpu.VMEM((CHUNK,), jnp.int32),
            pltpu.VMEM((CHUNK, D), jnp.float32),
            pltpu.VMEM((CHUNK, D), jnp.float32),
            pltpu.SemaphoreType.DMA,
            pltpu.SemaphoreType.DMA,
            pltpu.SemaphoreType.DMA,
            pltpu.SemaphoreType.DMA,
        ],
    )


def _sc_edge_agg(cur, packed3):
    return _sc_edge_agg_build()(cur, packed3)


def _pool_part(bt_ref, feat):
    b = bt_ref[0, 0, :]
    oh = (lax.broadcasted_iota(jnp.int32, (G, _R), 0) == b[None, :])
    return jnp.dot(oh.astype(jnp.float32), feat,
                   preferred_element_type=jnp.float32)


def _accum_pool(pool_ref, part):
    i = pl.program_id(0)

    @pl.when(i == 0)
    def _():
        pool_ref[...] = part

    @pl.when(i != 0)
    def _():
        pool_ref[...] = pool_ref[...] + part


def _embed_body(x_ref, w_ref, bt_ref, h_ref, pool_ref):
    h = jnp.dot(x_ref[...], w_ref[...], preferred_element_type=jnp.float32)
    h_ref[...] = h
    _accum_pool(pool_ref, _pool_part(bt_ref, h))


_embed_call = pl.pallas_call(
    _embed_body,
    grid=(_GRID,),
    in_specs=[
        pl.BlockSpec((_R, 32), lambda i: (i, 0)),
        pl.BlockSpec((32, D), lambda i: (0, 0)),
        pl.BlockSpec((1, 1, _R), lambda i: (i, 0, 0)),
    ],
    out_specs=[
        pl.BlockSpec((_R, D), lambda i: (i, 0)),
        pl.BlockSpec((G, D), lambda i: (0, 0)),
    ],
    out_shape=[
        jax.ShapeDtypeStruct((N, D), jnp.float32),
        jax.ShapeDtypeStruct((G, D), jnp.float32),
    ],
)


def _layer_body(a0_ref, a1_ref, w_ref, bias_ref, bt_ref, cur_ref, pool_ref):
    s = a0_ref[...] + a1_ref[...]
    cur = jnp.dot(s, w_ref[...], preferred_element_type=jnp.float32)
    cur = cur + bias_ref[...]
    cur_ref[...] = cur
    _accum_pool(pool_ref, _pool_part(bt_ref, cur))


_layer_call = pl.pallas_call(
    _layer_body,
    grid=(_GRID,),
    in_specs=[
        pl.BlockSpec((_R, D), lambda i: (i, 0)),
        pl.BlockSpec((_R, D), lambda i: (i, 0)),
        pl.BlockSpec((D, D), lambda i: (0, 0)),
        pl.BlockSpec((1, D), lambda i: (0, 0)),
        pl.BlockSpec((1, 1, _R), lambda i: (i, 0, 0)),
    ],
    out_specs=[
        pl.BlockSpec((_R, D), lambda i: (i, 0)),
        pl.BlockSpec((G, D), lambda i: (0, 0)),
    ],
    out_shape=[
        jax.ShapeDtypeStruct((N, D), jnp.float32),
        jax.ShapeDtypeStruct((G, D), jnp.float32),
    ],
)


def kernel(x, edge_index, batch, W_embed, Ws, bs):
    src = edge_index[0]
    dst = edge_index[1]
    pad = E_PAD - E
    pad_idx = jnp.arange(pad, dtype=jnp.int32)
    src_p = jnp.concatenate([src, pad_idx % N])
    dst_p = jnp.concatenate([dst, N + (pad_idx % 16)])
    packed3 = (src_p | (dst_p << 14)).reshape(NW, CH_PER_TILE, CHUNK)
    batch3 = batch.reshape(_GRID, 1, _R)

    h, p0 = _embed_call(x, W_embed, batch3)
    pools = [p0]
    cur = h
    for i in range(N_LAYERS):
        agg = _sc_edge_agg(cur, packed3)
        cur, p = _layer_call(agg[0], agg[1], Ws[i], bs[i].reshape(1, D),
                             batch3)
        pools.append(p)
    return jnp.concatenate(pools, axis=-1)


# speedup vs baseline: 1.1001x; 1.1001x over previous
import functools

import jax
import jax.numpy as jnp
from jax import lax
from jax.experimental import pallas as pl
from jax.experimental.pallas import tpu as pltpu
from jax.experimental.pallas import tpu_sc as plsc

N = 10000
E = 320000
G = 128
D = 128
N_LAYERS = 3

NW = 32
CHUNK = 128
CH_PER_TILE = 81
E_PAD = NW * CH_PER_TILE * CHUNK
N_ACC = N + 16
ROUNDS = CH_PER_TILE

_R = 1000
_GRID = N // _R

def _sc_body(cur, packed3, out, acc,
             ip0, ip1, ip2, sr0, sr1, sr2, ds0, ds1, ds2, bf0, bf1, bf2,
             smi0, smi1, smi2, smg0, smg1, smg2, sms0, sms1, sms2):
    cid = lax.axis_index("c")
    sid = lax.axis_index("s")
    wid = sid * 2 + cid
    rows_per_tile = 624
    tail_base = 16 * rows_per_tile
    base = sid * rows_per_tile

    IP = (ip0, ip1, ip2)
    SRC = (sr0, sr1, sr2)
    DST = (ds0, ds1, ds2)
    BUF = (bf0, bf1, bf2)
    SEMI = (smi0, smi1, smi2)
    SEMG = (smg0, smg1, smg2)
    SEMS = (sms0, sms1, sms2)

    def fire_idx(g, r):
        pltpu.async_copy(packed3.at[wid, r], IP[g], SEMI[g])

    def wait_idx(g):
        pltpu.make_async_copy(packed3.at[0, 0], IP[g], SEMI[g]).wait()

    def unpack(g):
        for i in range(CHUNK // 16):
            v = IP[g][pl.ds(i * 16, 16)]
            SRC[g][pl.ds(i * 16, 16)] = v & 0x3FFF
            DST[g][pl.ds(i * 16, 16)] = lax.shift_right_logical(v, 14)

    def fire_gather(g):
        pltpu.async_copy(cur.at[SRC[g]], BUF[g], SEMG[g])

    def wait_gather(g):
        pltpu.make_async_copy(cur.at[pl.ds(0, CHUNK)], BUF[g],
                              SEMG[g]).wait()

    def fire_scatter(g):
        pltpu.async_copy(BUF[g], acc.at[DST[g]], SEMS[g], add=True)

    def wait_scatter(g):
        pltpu.make_async_copy(cur.at[pl.ds(0, CHUNK)], BUF[g],
                              SEMS[g]).wait()

    fire_idx(0, 0)
    fire_idx(1, 1)
    fire_idx(2, 2)
    pltpu.sync_copy(cur.at[pl.ds(base, rows_per_tile)],
                    acc.at[pl.ds(base, rows_per_tile)])

    @pl.when(sid == 15)
    def _():
        pltpu.sync_copy(cur.at[pl.ds(tail_base, N - tail_base)],
                        acc.at[pl.ds(tail_base, N - tail_base)])

    wait_idx(0)
    unpack(0)
    fire_gather(0)
    fire_idx(0, 3)
    plsc.subcore_barrier()

    def body(o, carry):
        for t in range(3):
            r = o * 3 + t
            x = t
            y = (t + 1) % 3
            wait_gather(x)
            fire_scatter(x)

            def _drain_y():
                wait_scatter(y)

            if t < 2:
                pl.when(o > 0)(_drain_y)
            else:
                _drain_y()

            @pl.when(r + 1 <= ROUNDS - 1)
            def _():
                wait_idx(y)
                unpack(y)
                fire_gather(y)

            @pl.when(r + 4 <= ROUNDS - 1)
            def _():
                fire_idx(y, r + 4)
        return carry

    lax.fori_loop(0, ROUNDS // 3, body, 0)
    wait_scatter((ROUNDS - 2) % 3)
    wait_scatter((ROUNDS - 1) % 3)
    plsc.subcore_barrier()

    pltpu.sync_copy(acc.at[pl.ds(base, rows_per_tile)],
                    out.at[cid, pl.ds(base, rows_per_tile)])

    @pl.when(sid == 15)
    def _():
        pltpu.sync_copy(acc.at[pl.ds(tail_base, N - tail_base)],
                        out.at[cid, pl.ds(tail_base, N - tail_base)])


@functools.cache
def _sc_edge_agg_build():
    mesh = plsc.VectorSubcoreMesh(core_axis_name="c", subcore_axis_name="s")
    return pl.kernel(
        _sc_body,
        out_type=jax.ShapeDtypeStruct((2, N, D), jnp.float32),
        mesh=mesh,
        scratch_types=(
            [pltpu.VMEM_SHARED((N_ACC, D), jnp.float32)]
            + [pltpu.VMEM((CHUNK,), jnp.int32) for _ in range(9)]
            + [pltpu.VMEM((CHUNK, D), jnp.float32) for _ in range(3)]
            + [pltpu.SemaphoreType.DMA for _ in range(9)]
        ),
    )


def _sc_edge_agg(cur, packed3):
    return _sc_edge_agg_build()(cur, packed3)


def _pool_part(bt_ref, feat):
    b = bt_ref[0, 0, :]
    oh = (lax.broadcasted_iota(jnp.int32, (G, _R), 0) == b[None, :])
    return jnp.dot(oh.astype(jnp.float32), feat,
                   preferred_element_type=jnp.float32)


def _accum_pool(pool_ref, part):
    i = pl.program_id(0)

    @pl.when(i == 0)
    def _():
        pool_ref[...] = part

    @pl.when(i != 0)
    def _():
        pool_ref[...] = pool_ref[...] + part


def _embed_body(x_ref, w_ref, bt_ref, h_ref, pool_ref):
    h = jnp.dot(x_ref[...], w_ref[...], preferred_element_type=jnp.float32)
    h_ref[...] = h
    _accum_pool(pool_ref, _pool_part(bt_ref, h))


_embed_call = pl.pallas_call(
    _embed_body,
    grid=(_GRID,),
    in_specs=[
        pl.BlockSpec((_R, 32), lambda i: (i, 0)),
        pl.BlockSpec((32, D), lambda i: (0, 0)),
        pl.BlockSpec((1, 1, _R), lambda i: (i, 0, 0)),
    ],
    out_specs=[
        pl.BlockSpec((_R, D), lambda i: (i, 0)),
        pl.BlockSpec((G, D), lambda i: (0, 0)),
    ],
    out_shape=[
        jax.ShapeDtypeStruct((N, D), jnp.float32),
        jax.ShapeDtypeStruct((G, D), jnp.float32),
    ],
)


def _layer_body(a0_ref, a1_ref, w_ref, bias_ref, bt_ref, cur_ref, pool_ref):
    s = a0_ref[...] + a1_ref[...]
    cur = jnp.dot(s, w_ref[...], preferred_element_type=jnp.float32)
    cur = cur + bias_ref[...]
    cur_ref[...] = cur
    _accum_pool(pool_ref, _pool_part(bt_ref, cur))


_layer_call = pl.pallas_call(
    _layer_body,
    grid=(_GRID,),
    in_specs=[
        pl.BlockSpec((_R, D), lambda i: (i, 0)),
        pl.BlockSpec((_R, D), lambda i: (i, 0)),
        pl.BlockSpec((D, D), lambda i: (0, 0)),
        pl.BlockSpec((1, D), lambda i: (0, 0)),
        pl.BlockSpec((1, 1, _R), lambda i: (i, 0, 0)),
    ],
    out_specs=[
        pl.BlockSpec((_R, D), lambda i: (i, 0)),
        pl.BlockSpec((G, D), lambda i: (0, 0)),
    ],
    out_shape=[
        jax.ShapeDtypeStruct((N, D), jnp.float32),
        jax.ShapeDtypeStruct((G, D), jnp.float32),
    ],
)


def kernel(x, edge_index, batch, W_embed, Ws, bs):
    src = edge_index[0]
    dst = edge_index[1]
    pad = E_PAD - E
    pad_idx = jnp.arange(pad, dtype=jnp.int32)
    src_p = jnp.concatenate([src, pad_idx % N])
    dst_p = jnp.concatenate([dst, N + (pad_idx % 16)])
    packed3 = (src_p | (dst_p << 14)).reshape(NW, CH_PER_TILE, CHUNK)
    batch3 = batch.reshape(_GRID, 1, _R)

    h, p0 = _embed_call(x, W_embed, batch3)
    pools = [p0]
    cur = h
    for i in range(N_LAYERS):
        agg = _sc_edge_agg(cur, packed3)
        cur, p = _layer_call(agg[0], agg[1], Ws[i], bs[i].reshape(1, D),
                             batch3)
        pools.append(p)
    return jnp.concatenate(pools, axis=-1)
